# baseline (speedup 1.0000x reference)
import functools

import jax
import jax.numpy as jnp
from jax.experimental import pallas as pl
from jax.experimental.pallas import tpu as pltpu


_BM = 1024
_BF = 1024
_BD = 256
_KSPLIT = 2


def _moe_kernel(num_tokens, ni, nj, nd, num_experts, bf, kc,
                x_ref, wg_ref, bg_ref, w1_ref, b1_ref, w2_ref, b2_ref,
                out_ref, cnt_ref, aux_ref, h_ref, load_ref):
    i = pl.program_id(0)
    s = pl.program_id(1)

    @pl.when(s == 0)
    def _gating():
        scores = jax.lax.dot_general(
            x_ref[...], wg_ref[...], (((1,), (1,)), ((), ())),
            preferred_element_type=jnp.float32) + bg_ref[...]

        iota = jax.lax.broadcasted_iota(jnp.int32, scores.shape, 1)
        m1 = jnp.max(scores, axis=1, keepdims=True)
        i1 = jnp.min(jnp.where(scores == m1, iota, num_experts),
                     axis=1, keepdims=True)
        rest = jnp.where(iota == i1, -jnp.inf, scores)
        m2 = jnp.max(rest, axis=1, keepdims=True)
        i2 = jnp.min(jnp.where(rest == m2, iota, num_experts),
                     axis=1, keepdims=True)
        e = jnp.exp(m2 - m1)
        w2 = e / (1.0 + e)
        sel = (iota == i1).astype(jnp.int32) + \
              ((iota == i2) & (w2 > 0.0)).astype(jnp.int32)
        cnt_blk = jnp.sum(sel, axis=0, keepdims=True)
        load_blk = jnp.sum(scores, axis=0, keepdims=True)

        @pl.when(i == 0)
        def _():
            cnt_ref[...] = cnt_blk
            load_ref[...] = load_blk

        @pl.when(i > 0)
        def _():
            cnt_ref[...] = cnt_ref[...] + cnt_blk
            load_ref[...] = load_ref[...] + load_blk

    @pl.when(s < nj)
    def _phase1():
        h = jax.lax.dot_general(
            x_ref[...], w1_ref[...],
            (((1,), (0,)), ((), ())),
            preferred_element_type=jnp.float32) + b1_ref[...]
        h_ref[:, pl.ds(s * bf, bf)] = jnp.maximum(h, 0.0).astype(jnp.bfloat16)

    @pl.when(s >= nj)
    def _phase2():
        k = (s - nj) % _KSPLIT
        acc = jax.lax.dot_general(
            h_ref[:, pl.ds(k * kc, kc)], w2_ref[...],
            (((1,), (0,)), ((), ())),
            preferred_element_type=jnp.float32)

        @pl.when(k == 0)
        def _():
            out_ref[...] = acc + b2_ref[...]

        @pl.when(k > 0)
        def _():
            out_ref[...] = out_ref[...] + acc

    @pl.when((i == ni - 1) & (s == nj + _KSPLIT * nd - 1))
    def _finalize():
        load = load_ref[...] * (1.0 / num_tokens)
        aux_ref[...] = jnp.sum(load * load).reshape(1, 1)


def kernel(x, Wg, bg, W1, b1, W2, b2):
    B, S, d = x.shape
    num_tokens = B * S
    d_ff = W1.shape[1]
    num_experts = Wg.shape[1]
    x_flat = x.reshape(num_tokens, d).astype(jnp.bfloat16)

    bm = min(_BM, num_tokens)
    bf = min(_BF, d_ff)
    bd = min(_BD, d)
    ni = num_tokens // bm
    nj = d_ff // bf
    nd = d // bd
    kc = d_ff // _KSPLIT

    out, cnt, aux = pl.pallas_call(
        functools.partial(_moe_kernel, num_tokens, ni, nj, nd,
                          num_experts, bf, kc),
        grid=(ni, nj + _KSPLIT * nd),
        in_specs=[
            pl.BlockSpec((bm, d), lambda i, s: (i, 0)),
            pl.BlockSpec((num_experts, d), lambda i, s: (0, 0)),
            pl.BlockSpec((1, num_experts), lambda i, s: (0, 0)),
            pl.BlockSpec((d, bf), lambda i, s: (0, jnp.minimum(s, nj - 1))),
            pl.BlockSpec((1, bf), lambda i, s: (0, jnp.minimum(s, nj - 1))),
            pl.BlockSpec((kc, bd),
                         lambda i, s: (jnp.maximum(s - nj, 0) % _KSPLIT,
                                       jnp.maximum(s - nj, 0) // _KSPLIT)),
            pl.BlockSpec((1, bd),
                         lambda i, s: (0, jnp.maximum(s - nj, 0) // _KSPLIT)),
        ],
        out_specs=[
            pl.BlockSpec((bm, bd),
                         lambda i, s: (i, jnp.maximum(s - nj, 0) // _KSPLIT)),
            pl.BlockSpec((1, num_experts), lambda i, s: (0, 0)),
            pl.BlockSpec((1, 1), lambda i, s: (0, 0)),
        ],
        out_shape=[
            jax.ShapeDtypeStruct((num_tokens, d), jnp.float32),
            jax.ShapeDtypeStruct((1, num_experts), jnp.int32),
            jax.ShapeDtypeStruct((1, 1), jnp.float32),
        ],
        scratch_shapes=[
            pltpu.VMEM((bm, d_ff), jnp.bfloat16),
            pltpu.VMEM((1, num_experts), jnp.float32),
        ],
        compiler_params=pltpu.CompilerParams(
            dimension_semantics=("arbitrary", "arbitrary"),
        ),
    )(x_flat, Wg.T, bg.reshape(1, num_experts), W1, b1.reshape(1, d_ff),
      W2, b2.reshape(1, d))

    return (out.reshape(B, S, d), aux[0, 0], cnt.reshape(num_experts))


# speedup vs baseline: 1.0110x; 1.0110x over previous
import functools

import jax
import jax.numpy as jnp
from jax.experimental import pallas as pl
from jax.experimental.pallas import tpu as pltpu


_BM = 1024
_BF = 512
_BD = 256
_KSPLIT = 2


def _moe_kernel(num_tokens, ni, nj, nd, num_experts, bf, kc,
                x_ref, wg_ref, bg_ref, w1_ref, b1_ref, w2_ref, b2_ref,
                out_ref, cnt_ref, aux_ref, h_ref, load_ref):
    i = pl.program_id(0)
    s = pl.program_id(1)

    @pl.when(s == 0)
    def _gating():
        scores = jax.lax.dot_general(
            x_ref[...], wg_ref[...], (((1,), (0,)), ((), ())),
            preferred_element_type=jnp.float32) + bg_ref[...]

        iota = jax.lax.broadcasted_iota(jnp.int32, scores.shape, 1)
        m1 = jnp.max(scores, axis=1, keepdims=True)
        i1 = jnp.min(jnp.where(scores == m1, iota, num_experts),
                     axis=1, keepdims=True)
        rest = jnp.where(iota == i1, -jnp.inf, scores)
        m2 = jnp.max(rest, axis=1, keepdims=True)
        i2 = jnp.min(jnp.where(rest == m2, iota, num_experts),
                     axis=1, keepdims=True)
        e = jnp.exp(m2 - m1)
        w2 = e / (1.0 + e)
        sel = (iota == i1).astype(jnp.int32) + \
              ((iota == i2) & (w2 > 0.0)).astype(jnp.int32)
        cnt_blk = jnp.sum(sel, axis=0, keepdims=True)
        load_blk = jnp.sum(scores, axis=0, keepdims=True)

        @pl.when(i == 0)
        def _():
            cnt_ref[...] = cnt_blk
            load_ref[...] = load_blk

        @pl.when(i > 0)
        def _():
            cnt_ref[...] = cnt_ref[...] + cnt_blk
            load_ref[...] = load_ref[...] + load_blk

    @pl.when(s < nj)
    def _phase1():
        h = jax.lax.dot_general(
            x_ref[...], w1_ref[...],
            (((1,), (0,)), ((), ())),
            preferred_element_type=jnp.float32) + b1_ref[...]
        h_ref[:, pl.ds(s * bf, bf)] = jnp.maximum(h, 0.0).astype(jnp.bfloat16)

    @pl.when(s >= nj)
    def _phase2():
        k = (s - nj) % _KSPLIT
        acc = jax.lax.dot_general(
            h_ref[:, pl.ds(k * kc, kc)], w2_ref[...],
            (((1,), (0,)), ((), ())),
            preferred_element_type=jnp.float32)

        @pl.when(k == 0)
        def _():
            out_ref[...] = acc + b2_ref[...]

        @pl.when(k > 0)
        def _():
            out_ref[...] = out_ref[...] + acc

    @pl.when((i == ni - 1) & (s == nj + _KSPLIT * nd - 1))
    def _finalize():
        load = load_ref[...] * (1.0 / num_tokens)
        aux_ref[...] = jnp.sum(load * load).reshape(1, 1)


def kernel(x, Wg, bg, W1, b1, W2, b2):
    B, S, d = x.shape
    num_tokens = B * S
    d_ff = W1.shape[1]
    num_experts = Wg.shape[1]
    x_flat = x.reshape(num_tokens, d)

    bm = min(_BM, num_tokens)
    bf = min(_BF, d_ff)
    bd = min(_BD, d)
    ni = num_tokens // bm
    nj = d_ff // bf
    nd = d // bd
    kc = d_ff // _KSPLIT

    out, cnt, aux = pl.pallas_call(
        functools.partial(_moe_kernel, num_tokens, ni, nj, nd,
                          num_experts, bf, kc),
        grid=(ni, nj + _KSPLIT * nd),
        in_specs=[
            pl.BlockSpec((bm, d), lambda i, s: (i, 0)),
            pl.BlockSpec((d, num_experts), lambda i, s: (0, 0)),
            pl.BlockSpec((1, num_experts), lambda i, s: (0, 0)),
            pl.BlockSpec((d, bf), lambda i, s: (0, jnp.minimum(s, nj - 1))),
            pl.BlockSpec((1, bf), lambda i, s: (0, jnp.minimum(s, nj - 1))),
            pl.BlockSpec((kc, bd),
                         lambda i, s: (jnp.maximum(s - nj, 0) % _KSPLIT,
                                       jnp.maximum(s - nj, 0) // _KSPLIT)),
            pl.BlockSpec((1, bd),
                         lambda i, s: (0, jnp.maximum(s - nj, 0) // _KSPLIT)),
        ],
        out_specs=[
            pl.BlockSpec((bm, bd),
                         lambda i, s: (i, jnp.maximum(s - nj, 0) // _KSPLIT)),
            pl.BlockSpec((1, num_experts), lambda i, s: (0, 0)),
            pl.BlockSpec((1, 1), lambda i, s: (0, 0)),
        ],
        out_shape=[
            jax.ShapeDtypeStruct((num_tokens, d), jnp.float32),
            jax.ShapeDtypeStruct((1, num_experts), jnp.int32),
            jax.ShapeDtypeStruct((1, 1), jnp.float32),
        ],
        scratch_shapes=[
            pltpu.VMEM((bm, d_ff), jnp.bfloat16),
            pltpu.VMEM((1, num_experts), jnp.float32),
        ],
        compiler_params=pltpu.CompilerParams(
            dimension_semantics=("arbitrary", "arbitrary"),
        ),
    )(x_flat, Wg, bg.reshape(1, num_experts), W1, b1.reshape(1, d_ff),
      W2, b2.reshape(1, d))

    return (out.reshape(B, S, d), aux[0, 0], cnt.reshape(num_experts))
